# Initial kernel scaffold; baseline (speedup 1.0000x reference)
import jax
import jax.numpy as jnp
from jax.experimental import pallas as pl


def kernel(x, table):
    raise NotImplementedError("write your pallas kernel here")


# speedup vs baseline: 1.4573x; 1.4573x over previous
import functools
import jax
import jax.numpy as jnp
from jax import lax
from jax.experimental import pallas as pl
from jax.experimental.pallas import tpu as pltpu
from jax.experimental.pallas import tpu_sc as plsc

_NW = 32
_IDXW = 128
_K = 8


def _make_gather(n_rows, vocab, emb):
    rows_per_w = n_rows // _NW
    n_chunks = rows_per_w // _K
    chunk_rows = _K * _IDXW

    mesh = plsc.VectorSubcoreMesh(core_axis_name="c", subcore_axis_name="s")

    @functools.partial(
        pl.kernel,
        mesh=mesh,
        out_type=jax.ShapeDtypeStruct((n_rows * _IDXW, emb), jnp.float32),
        scratch_types=[
            pltpu.VMEM((_K, _IDXW), jnp.int32),
            pltpu.VMEM((chunk_rows, emb), jnp.float32),
            pltpu.SemaphoreType.DMA,
        ],
        compiler_params=pltpu.CompilerParams(use_tc_tiling_on_sc=False),
    )
    def gather_kernel(idx_hbm, table_hbm, out_hbm, idx_v, rows_v, sem):
        wid = lax.axis_index("s") * 2 + lax.axis_index("c")
        base_row = wid * rows_per_w

        def body(c, carry):
            r = base_row + c * _K
            pltpu.sync_copy(idx_hbm.at[pl.ds(r, _K)], idx_v)
            copies = [
                pltpu.async_copy(
                    table_hbm.at[idx_v.at[j]],
                    rows_v.at[pl.ds(j * _IDXW, _IDXW)],
                    sem,
                )
                for j in range(_K)
            ]
            for cp in copies:
                cp.wait()
            pltpu.sync_copy(rows_v, out_hbm.at[pl.ds(r * _IDXW, chunk_rows)])
            return carry

        lax.fori_loop(0, n_chunks, body, 0)

    return gather_kernel


def kernel(x, table):
    b, l = x.shape
    vocab, emb = table.shape
    n = b * l
    xf = x.reshape(n // _IDXW, _IDXW).astype(jnp.int32)
    gathered = _make_gather(n // _IDXW, vocab, emb)(xf, table)
    return gathered.reshape(b, l, emb)


# speedup vs baseline: 1.4824x; 1.0172x over previous
import functools
import jax
import jax.numpy as jnp
from jax import lax
from jax.experimental import pallas as pl
from jax.experimental.pallas import tpu as pltpu
from jax.experimental.pallas import tpu_sc as plsc

_NW = 32
_IDXW = 128
_K = 8
_NBUF = 3


def _make_gather(n_rows, vocab, emb):
    rows_per_w = n_rows // _NW
    n_chunks = rows_per_w // _K
    n_groups = n_chunks // _NBUF
    chunk_rows = _K * _IDXW

    mesh = plsc.VectorSubcoreMesh(core_axis_name="c", subcore_axis_name="s")

    @functools.partial(
        pl.kernel,
        mesh=mesh,
        out_type=jax.ShapeDtypeStruct((n_rows * _IDXW, emb), jnp.float32),
        scratch_types=[
            pltpu.VMEM((_NBUF, _K, _IDXW), jnp.int32),
            pltpu.VMEM((_NBUF, chunk_rows, emb), jnp.float32),
            pltpu.SemaphoreType.DMA((_NBUF,)),
            pltpu.SemaphoreType.DMA((_NBUF,)),
        ],
        compiler_params=pltpu.CompilerParams(use_tc_tiling_on_sc=False),
    )
    def gather_kernel(idx_hbm, table_hbm, out_hbm, idx_v, rows_v, gsem, osem):
        wid = lax.axis_index("s") * 2 + lax.axis_index("c")
        base_row = wid * rows_per_w

        def fire_chunk(c, b):
            r = base_row + c * _K
            pltpu.sync_copy(idx_hbm.at[pl.ds(r, _K)], idx_v.at[b])
            for j in range(_K):
                pltpu.async_copy(
                    table_hbm.at[idx_v.at[b].at[j]],
                    rows_v.at[b].at[pl.ds(j * _IDXW, _IDXW)],
                    gsem.at[b],
                )

        def wait_gathers(b):
            pltpu.make_async_copy(
                table_hbm.at[pl.ds(0, chunk_rows)], rows_v.at[b], gsem.at[b]
            ).wait()

        def fire_out(c, b):
            r = base_row + c * _K
            pltpu.async_copy(
                rows_v.at[b], out_hbm.at[pl.ds(r * _IDXW, chunk_rows)], osem.at[b]
            )

        def wait_out(b):
            pltpu.make_async_copy(
                rows_v.at[b], out_hbm.at[pl.ds(0, chunk_rows)], osem.at[b]
            ).wait()

        fire_chunk(0, 0)
        for s in range(1, _NBUF):
            wait_gathers((s - 1) % _NBUF)
            fire_out(s - 1, (s - 1) % _NBUF)
            fire_chunk(s, s % _NBUF)

        n_groups = (n_chunks - _NBUF) // _NBUF

        @pl.loop(0, n_groups)
        def _(g):
            for i in range(_NBUF):
                s = _NBUF + g * _NBUF + i
                bp = (i - 1) % _NBUF
                wait_gathers(bp)
                fire_out(s - 1, bp)
                wait_out(i)
                fire_chunk(s, i)

        for s in range(_NBUF + n_groups * _NBUF, n_chunks):
            b = s % _NBUF
            bp = (s - 1) % _NBUF
            wait_gathers(bp)
            fire_out(s - 1, bp)
            wait_out(b)
            fire_chunk(s, b)

        s_last = n_chunks - 1
        bl = s_last % _NBUF
        wait_gathers(bl)
        fire_out(s_last, bl)
        for i in range(_NBUF):
            wait_out((n_chunks - _NBUF + i) % _NBUF)

    return gather_kernel


def kernel(x, table):
    b, l = x.shape
    vocab, emb = table.shape
    n = b * l
    xf = x.reshape(n // _IDXW, _IDXW).astype(jnp.int32)
    gathered = _make_gather(n // _IDXW, vocab, emb)(xf, table)
    return gathered.reshape(b, l, emb)
